# Initial kernel scaffold; baseline (speedup 1.0000x reference)
import jax
import jax.numpy as jnp
from jax.experimental import pallas as pl


def kernel(x, edge_index, edge_attr, batch, enc_W, enc_b, Wq0, bq0, Wk0, bk0, Wv0, bv0, We0, Wskip0, bskip0, Wbeta0, gamma0, beta0, Wq1, bq1, Wk1, bk1, Wv1, bv1, We1, Wskip1, bskip1, Wbeta1, gamma1, beta1, dec_W, dec_b, out_bias):
    raise NotImplementedError("write your pallas kernel here")


# speedup vs baseline: 1.0484x; 1.0484x over previous
import jax
import jax.numpy as jnp
from jax.experimental import pallas as pl

N = 50000
E = 800000
D = 128
H = 4
C = D // H
G = 16
EPS = 1e-5


def _conv(x, src, dst, edge_attr, Wq, bq, Wk, bk, Wv, bv, We, Wskip, bskip, Wbeta):
    q = (x @ Wq + bq).reshape(-1, H, C)
    k = (x @ Wk + bk).reshape(-1, H, C)
    v = (x @ Wv + bv).reshape(-1, H, C)
    e = (edge_attr @ We).reshape(-1, H, C)
    k_j = k[src] + e
    q_i = q[dst]
    alpha = (q_i * k_j).sum(-1) / jnp.sqrt(float(C))
    ex = jnp.exp(alpha)
    den = jax.ops.segment_sum(ex, dst, num_segments=N)
    msg = (v[src] + e) * ex[:, :, None]
    num = jax.ops.segment_sum(msg, dst, num_segments=N).reshape(-1, D)
    out = num / (jnp.repeat(den, C, axis=1) + 1e-16)
    x_r = x @ Wskip + bskip
    b = jax.nn.sigmoid(jnp.concatenate([out, x_r, out - x_r], axis=-1) @ Wbeta)
    return b * x_r + (1.0 - b) * out


def kernel(x, edge_index, edge_attr, batch, enc_W, enc_b,
           Wq0, bq0, Wk0, bk0, Wv0, bv0, We0, Wskip0, bskip0, Wbeta0, gamma0, beta0,
           Wq1, bq1, Wk1, bk1, Wv1, bv1, We1, Wskip1, bskip1, Wbeta1, gamma1, beta1,
           dec_W, dec_b, out_bias):
    kw = dict(locals())
    src = edge_index[0]
    dst = edge_index[1]
    h = x @ enc_W + enc_b
    for l in range(2):
        h = _conv(h, src, dst, edge_attr, kw["Wq%d" % l], kw["bq%d" % l],
                  kw["Wk%d" % l], kw["bk%d" % l], kw["Wv%d" % l], kw["bv%d" % l],
                  kw["We%d" % l], kw["Wskip%d" % l], kw["bskip%d" % l], kw["Wbeta%d" % l])
        mean = h.mean(axis=0)
        var = h.var(axis=0)
        h = (h - mean) / jnp.sqrt(var + EPS) * kw["gamma%d" % l] + kw["beta%d" % l]
        h = jax.nn.elu(h)
    cnt = jax.ops.segment_sum(jnp.ones((h.shape[0],), jnp.float32), batch, num_segments=G)
    h_g = jax.ops.segment_sum(h, batch, num_segments=G) / jnp.maximum(cnt, 1.0)[:, None]
    return h_g @ dec_W + dec_b + out_bias


# speedup vs baseline: 16.0464x; 15.3058x over previous
import jax
import jax.numpy as jnp
from jax import lax
from jax.experimental import pallas as pl
from jax.experimental.pallas import tpu as pltpu
from jax.experimental.pallas import tpu_sc as plsc

N = 50000
E = 800000
D = 128
H = 4
C = D // H
G = 16
EPS = 1e-5

NCHUNK = 10
CHUNK = 5120
NPAD = NCHUNK * CHUNK
TILE_ROWS = CHUNK // 16
DSROWS = CHUNK // 16
NW = 32
E_TILE = E // NW
B_SCAN = 512
N_SCAN = (E_TILE + B_SCAN - 1) // B_SCAN
EPADV = 800128
ZCAP = 3200
ZTOT = NW * NCHUNK * ZCAP
ZPAD = ZTOT + 16
B_PROC = 64
QW = 2 * D

BLK = 1000
NBLK = N // BLK

_SC_PARAMS = pltpu.CompilerParams(needs_layout_passes=False)


def _pre_body(h_ref, A_ref, ba_ref, B_ref, bb_ref, Ws_ref, bs_ref,
              qaug_ref, kv_ref, xr_ref):
    h = h_ref[...]
    qaug_ref[...] = jnp.dot(h, A_ref[...], preferred_element_type=jnp.float32) + ba_ref[...]
    kv_ref[...] = jnp.dot(h, B_ref[...], preferred_element_type=jnp.float32) + bb_ref[...]
    xr_ref[...] = jnp.dot(h, Ws_ref[...], preferred_element_type=jnp.float32) + bs_ref[...]


def _tc_pre(h, A, ba, B, bb, Ws, bs):
    din = h.shape[1]
    return pl.pallas_call(
        _pre_body,
        grid=(NBLK,),
        in_specs=[
            pl.BlockSpec((BLK, din), lambda i: (i, 0)),
            pl.BlockSpec((din, QW), lambda i: (0, 0)),
            pl.BlockSpec((1, QW), lambda i: (0, 0)),
            pl.BlockSpec((din, 2 * D), lambda i: (0, 0)),
            pl.BlockSpec((1, 2 * D), lambda i: (0, 0)),
            pl.BlockSpec((din, D), lambda i: (0, 0)),
            pl.BlockSpec((1, D), lambda i: (0, 0)),
        ],
        out_specs=[
            pl.BlockSpec((BLK, QW), lambda i: (i, 0)),
            pl.BlockSpec((BLK, 2 * D), lambda i: (i, 0)),
            pl.BlockSpec((BLK, D), lambda i: (i, 0)),
        ],
        out_shape=[
            jax.ShapeDtypeStruct((N, QW), jnp.float32),
            jax.ShapeDtypeStruct((N, 2 * D), jnp.float32),
            jax.ShapeDtypeStruct((N, D), jnp.float32),
        ],
    )(h, A, ba, B, bb, Ws, bs)


def _post1_body(acc_ref, ds_ref, xr_ref, selden_ref, sels_ref, wba_ref, wbb_ref,
                h_ref, stats_ref):
    i = pl.program_id(0)
    acc = acc_ref[...]
    ds8 = ds_ref[...]
    xr = xr_ref[...]
    den = jnp.dot(ds8, selden_ref[...], preferred_element_type=jnp.float32)
    sterm = jnp.dot(ds8, sels_ref[...], preferred_element_type=jnp.float32)
    num = acc + sterm
    out = num / (den + 1e-16)
    logit = (jnp.sum(out * wba_ref[...], axis=1, keepdims=True)
             + jnp.sum(xr * wbb_ref[...], axis=1, keepdims=True))
    b = jax.nn.sigmoid(logit)
    h = b * xr + (1.0 - b) * out
    h_ref[...] = h
    sh = jnp.sum(h, axis=0, keepdims=True)
    sh2 = jnp.sum(h * h, axis=0, keepdims=True)
    rows = jnp.concatenate([sh, sh2, jnp.zeros((6, D), jnp.float32)], axis=0)

    @pl.when(i == 0)
    def _():
        stats_ref[...] = jnp.zeros_like(stats_ref)

    stats_ref[...] += rows


def _tc_post1(acc, ds8, xr, selden, sels, wba, wbb):
    return pl.pallas_call(
        _post1_body,
        grid=(NBLK,),
        in_specs=[
            pl.BlockSpec((BLK, D), lambda i: (i, 0)),
            pl.BlockSpec((BLK, 8), lambda i: (i, 0)),
            pl.BlockSpec((BLK, D), lambda i: (i, 0)),
            pl.BlockSpec((8, D), lambda i: (0, 0)),
            pl.BlockSpec((8, D), lambda i: (0, 0)),
            pl.BlockSpec((1, D), lambda i: (0, 0)),
            pl.BlockSpec((1, D), lambda i: (0, 0)),
        ],
        out_specs=[
            pl.BlockSpec((BLK, D), lambda i: (i, 0)),
            pl.BlockSpec((8, D), lambda i: (0, 0)),
        ],
        out_shape=[
            jax.ShapeDtypeStruct((N, D), jnp.float32),
            jax.ShapeDtypeStruct((8, D), jnp.float32),
        ],
    )(acc, ds8, xr, selden, sels, wba, wbb)


def _norm(h, stats_ref, gamma_ref, beta_ref):
    mean = stats_ref[0:1, :] * (1.0 / N)
    ex2 = stats_ref[1:2, :] * (1.0 / N)
    var = ex2 - mean * mean
    inv = lax.rsqrt(var + EPS)
    hn = (h - mean) * inv * gamma_ref[...] + beta_ref[...]
    return jnp.where(hn > 0, hn, jnp.exp(jnp.minimum(hn, 0.0)) - 1.0)


def _post2a_body(h_ref, stats_ref, gamma_ref, beta_ref, hn_ref):
    hn_ref[...] = _norm(h_ref[...], stats_ref, gamma_ref, beta_ref)


def _tc_post2_mid(h, stats, gamma, beta):
    return pl.pallas_call(
        _post2a_body,
        grid=(NBLK,),
        in_specs=[
            pl.BlockSpec((BLK, D), lambda i: (i, 0)),
            pl.BlockSpec((8, D), lambda i: (0, 0)),
            pl.BlockSpec((1, D), lambda i: (0, 0)),
            pl.BlockSpec((1, D), lambda i: (0, 0)),
        ],
        out_specs=pl.BlockSpec((BLK, D), lambda i: (i, 0)),
        out_shape=jax.ShapeDtypeStruct((N, D), jnp.float32),
    )(h, stats, gamma, beta)


def _post2b_body(h_ref, stats_ref, gamma_ref, beta_ref, batch_ref,
                 pooled_ref, cnts_ref):
    i = pl.program_id(0)
    hn = _norm(h_ref[...], stats_ref, gamma_ref, beta_ref)
    bids = batch_ref[0, 0, :]
    gids = lax.broadcasted_iota(jnp.int32, (1, G), 1)
    onehot = (bids[:, None] == gids).astype(jnp.float32)
    psum = lax.dot_general(onehot, hn, (((0,), (0,)), ((), ())),
                           preferred_element_type=jnp.float32)
    crow = jnp.sum(onehot, axis=0)
    cblk = jnp.broadcast_to(crow[:, None], (G, D))

    @pl.when(i == 0)
    def _():
        pooled_ref[...] = jnp.zeros_like(pooled_ref)
        cnts_ref[...] = jnp.zeros_like(cnts_ref)

    pooled_ref[...] += psum
    cnts_ref[...] += cblk


def _tc_post2_final(h, stats, gamma, beta, batch3d):
    return pl.pallas_call(
        _post2b_body,
        grid=(NBLK,),
        in_specs=[
            pl.BlockSpec((BLK, D), lambda i: (i, 0)),
            pl.BlockSpec((8, D), lambda i: (0, 0)),
            pl.BlockSpec((1, D), lambda i: (0, 0)),
            pl.BlockSpec((1, D), lambda i: (0, 0)),
            pl.BlockSpec((1, 1, BLK), lambda i: (i, 0, 0)),
        ],
        out_specs=[
            pl.BlockSpec((G, D), lambda i: (0, 0)),
            pl.BlockSpec((G, D), lambda i: (0, 0)),
        ],
        out_shape=[
            jax.ShapeDtypeStruct((G, D), jnp.float32),
            jax.ShapeDtypeStruct((G, D), jnp.float32),
        ],
    )(h, stats, gamma, beta, batch3d)


def _dec_body(pooled_ref, cnts_ref, w_ref, b_ref, y_ref):
    hg = pooled_ref[...] / jnp.maximum(cnts_ref[...], 1.0)
    y_ref[...] = jnp.dot(hg, w_ref[...], preferred_element_type=jnp.float32) + b_ref[...]


def _tc_dec(pooled, cnts, wp, bp):
    return pl.pallas_call(
        _dec_body,
        out_shape=jax.ShapeDtypeStruct((G, 8), jnp.float32),
    )(pooled, cnts, wp, bp)


def _sc_bucket_body(src, dst, attr, bpk, bat, bcnt,
                    stag_src, stag_dst, stag_attr, cb_pk, cb_at, pfx, cvec):
    cid = lax.axis_index("c")
    sid = lax.axis_index("s")
    wid = cid * 16 + sid
    ebase = wid * E_TILE
    iota16 = lax.iota(jnp.int32, 16)
    sixteen = jnp.full((16,), 16, jnp.int32)

    for c in range(NCHUNK):
        lo = c * CHUNK

        def _scan_blk(blk, m):
            off = ebase + blk * B_SCAN
            pltpu.sync_copy(src.at[pl.ds(off, B_SCAN)], stag_src)
            pltpu.sync_copy(dst.at[pl.ds(off, B_SCAN)], stag_dst)
            pltpu.sync_copy(attr.at[pl.ds(off, B_SCAN)], stag_attr)
            lid0 = blk * B_SCAN

            def _grp(g, m):
                dv = stag_dst[pl.ds(g * 16, 16)]
                sv = stag_src[pl.ds(g * 16, 16)]
                dl = dv - lo
                lid = lid0 + g * 16 + iota16
                valid = (dl >= 0) & (dl < CHUNK) & (lid < E_TILE)
                ones = jnp.where(valid, 1, 0)
                pfx[pl.ds(0, 16)] = ones
                s = ones
                for stp in (1, 2, 4, 8):
                    gsh = plsc.load_gather(pfx, [jnp.maximum(iota16 - stp, 0)])
                    s = s + jnp.where(iota16 >= stp, gsh, 0)
                    pfx[pl.ds(0, 16)] = s
                excl = s - ones
                total = s[15]
                dest = jnp.where(valid, jnp.minimum(m + excl, ZCAP), ZCAP)
                packed = sv | lax.shift_left(dv, sixteen)
                plsc.store_scatter(cb_pk, [dest], packed)
                plsc.store_scatter(cb_at, [dest], stag_attr[pl.ds(g * 16, 16)])
                return m + total

            return lax.fori_loop(0, B_SCAN // 16, _grp, m)

        m = lax.fori_loop(0, N_SCAN, _scan_blk, jnp.int32(0))

        zb = (wid * NCHUNK + c) * ZCAP
        for kk in range(ZCAP // 1600):
            pltpu.sync_copy(cb_pk.at[pl.ds(kk * 1600, 1600)],
                            bpk.at[pl.ds(zb + kk * 1600, 1600)])
            pltpu.sync_copy(cb_at.at[pl.ds(kk * 1600, 1600)],
                            bat.at[pl.ds(zb + kk * 1600, 1600)])
        mc = jnp.minimum(m, ZCAP)
        if c == 0:
            cvec[pl.ds(0, 16)] = jnp.where(iota16 == c, mc, 0)
        else:
            cvec[pl.ds(0, 16)] = jnp.where(iota16 == c, mc, cvec[pl.ds(0, 16)])

    pltpu.sync_copy(cvec, bcnt.at[pl.ds(wid * 16, 16)])


def _sc_bucket(src, dst, attr):
    fn = pl.kernel(
        _sc_bucket_body,
        out_type=(
            jax.ShapeDtypeStruct((ZPAD,), jnp.int32),
            jax.ShapeDtypeStruct((ZPAD,), jnp.float32),
            jax.ShapeDtypeStruct((NW * 16,), jnp.int32),
        ),
        compiler_params=_SC_PARAMS,
        mesh=plsc.VectorSubcoreMesh(core_axis_name="c", subcore_axis_name="s"),
        scratch_types=[
            pltpu.VMEM((B_SCAN,), jnp.int32),
            pltpu.VMEM((B_SCAN,), jnp.int32),
            pltpu.VMEM((B_SCAN,), jnp.float32),
            pltpu.VMEM((ZCAP + 16,), jnp.int32),
            pltpu.VMEM((ZCAP + 16,), jnp.float32),
            pltpu.VMEM((16,), jnp.int32),
            pltpu.VMEM((16,), jnp.int32),
        ],
    )
    return fn(src, dst, attr)


def _sc_edge_body(qaug, kv, bpk, bat, bcnt, outn, outds,
                  cnt_v, pk_s, at_s, qidx, kvidx, dloc_r, wbuf,
                  q_rows, kv_rows, msg, dsbuf, dsidx, accn_sp, accds_sp,
                  semq, semkv):
    cid = lax.axis_index("c")
    sid = lax.axis_index("s")
    rbase = sid * TILE_ROWS
    dbase = sid * 64
    iota16 = lax.iota(jnp.int32, 16)
    sixteen = jnp.full((16,), 16, jnp.int32)
    zero16 = jnp.zeros((16,), jnp.float32)

    for k in range(DSROWS // 16):
        dsidx[pl.ds(k * 16, 16)] = k * 16 + iota16

    pltpu.sync_copy(bcnt.at[pl.ds(sid * 32, 32)], cnt_v)

    for chunk_i in range(NCHUNK // 2):
        chunk = (NCHUNK // 2) * cid + chunk_i
        lo = chunk * CHUNK

        def _zrow(i, carry):
            for jj in range(D // 16):
                msg[i, pl.ds(jj * 16, 16)] = zero16
            return carry
        lax.fori_loop(0, B_PROC, _zrow, 0)
        for kk in range(TILE_ROWS // B_PROC):
            pltpu.sync_copy(msg, accn_sp.at[pl.ds(rbase + kk * B_PROC, B_PROC)])
        rem = TILE_ROWS % B_PROC
        if rem:
            pltpu.sync_copy(msg.at[pl.ds(0, rem)],
                            accn_sp.at[pl.ds(rbase + (TILE_ROWS // B_PROC) * B_PROC, rem)])
        @pl.when(sid < DSROWS // 64)
        def _():
            pltpu.sync_copy(msg.at[pl.ds(0, 64)],
                            accds_sp.at[pl.ds(dbase, 64)])

        def _zds(i, carry):
            for jj in range(D // 16):
                dsbuf[i, pl.ds(jj * 16, 16)] = zero16
            return carry
        lax.fori_loop(0, DSROWS, _zds, 0)
        plsc.subcore_barrier()

        for z in range(2):
            w = sid * 2 + z
            zb = (w * NCHUNK + chunk) * ZCAP
            mvec = plsc.load_gather(cnt_v, [jnp.full((16,), z * 16, jnp.int32) + chunk])
            m = mvec[0]
            nblk = (m + (B_PROC - 1)) // B_PROC

            def _proc(blk, carry):
                base = blk * B_PROC
                pltpu.sync_copy(bpk.at[pl.ds(zb + base, B_PROC)], pk_s)
                pltpu.sync_copy(bat.at[pl.ds(zb + base, B_PROC)], at_s)
                for g in range(B_PROC // 16):
                    lgid = base + g * 16 + iota16
                    valid = lgid < m
                    pk = pk_s[pl.ds(g * 16, 16)]
                    sidv = jnp.where(valid, pk & 0xFFFF, 0)
                    didv = jnp.where(valid, lax.shift_right_logical(pk, sixteen), 0)
                    dlv = jnp.where(valid, didv - lo, 0)
                    qidx[pl.ds(g * 16, 16)] = didv
                    kvidx[pl.ds(g * 16, 16)] = sidv
                    dloc_r[pl.ds(g * 16, 16)] = dlv

                cq = pltpu.async_copy(qaug.at[qidx], q_rows, semq)
                ck = pltpu.async_copy(kv.at[kvidx], kv_rows, semkv)
                cq.wait()
                ck.wait()

                for g in range(B_PROC // 16):
                    rows16 = g * 16 + iota16
                    lgid = base + g * 16 + iota16
                    valid = lgid < m
                    av = jnp.where(valid, at_s[pl.ds(g * 16, 16)], 0.0)
                    dlv = dloc_r[pl.ds(g * 16, 16)]
                    dsrow = lax.shift_right_logical(dlv, jnp.full((16,), 4, jnp.int32))
                    dscol0 = lax.shift_left(dlv & 15, jnp.full((16,), 3, jnp.int32))
                    for h in range(H):
                        col0 = jnp.full((16,), h * C, jnp.int32)

                        def _dotc(cc, acc2):
                            cols = col0 + cc
                            qv = plsc.load_gather(q_rows, [rows16, cols])
                            kvv = plsc.load_gather(kv_rows, [rows16, cols])
                            return acc2 + qv * kvv

                        accv = lax.fori_loop(0, C, _dotc, zero16)
                        qe = plsc.load_gather(
                            q_rows, [rows16, jnp.full((16,), D + h, jnp.int32)])
                        alpha = accv + av * qe
                        wv = jnp.where(valid, jnp.exp(alpha), 0.0)
                        wbuf[h, pl.ds(g * 16, 16)] = wv
                        plsc.addupdate_scatter(
                            dsbuf, [dsrow, dscol0 + h], wv)
                        plsc.addupdate_scatter(
                            dsbuf, [dsrow, dscol0 + (H + h)], wv * av)

                def _medge(e, carry2):
                    e16 = jnp.full((16,), e, jnp.int32)
                    for j in range(D // 16):
                        h = j // 2
                        if j % 2 == 0:
                            wj = plsc.load_gather(
                                wbuf, [jnp.full((16,), h, jnp.int32), e16])
                        vj = kv_rows[e, pl.ds(D + j * 16, 16)]
                        msg[e, pl.ds(j * 16, 16)] = wj * vj
                    return carry2

                lax.fori_loop(0, B_PROC, _medge, 0)

                pltpu.sync_copy(msg, accn_sp.at[dloc_r], add=True)
                return carry

            lax.fori_loop(0, nblk, _proc, 0)

        pltpu.sync_copy(dsbuf, accds_sp.at[dsidx], add=True)
        plsc.subcore_barrier()

        obase = chunk * CHUNK + rbase
        for kk in range(TILE_ROWS // B_PROC):
            pltpu.sync_copy(accn_sp.at[pl.ds(rbase + kk * B_PROC, B_PROC)],
                            outn.at[pl.ds(obase + kk * B_PROC, B_PROC)])
        if rem:
            pltpu.sync_copy(
                accn_sp.at[pl.ds(rbase + (TILE_ROWS // B_PROC) * B_PROC, rem)],
                outn.at[pl.ds(obase + (TILE_ROWS // B_PROC) * B_PROC, rem)])
        @pl.when(sid < DSROWS // 64)
        def _():
            pltpu.sync_copy(accds_sp.at[pl.ds(dbase, 64)],
                            outds.at[pl.ds(chunk * DSROWS + dbase, 64)])
        plsc.subcore_barrier()


def _sc_edge(qaug, kv, bpk, bat, bcnt):
    fn = pl.kernel(
        _sc_edge_body,
        out_type=(
            jax.ShapeDtypeStruct((NPAD, D), jnp.float32),
            jax.ShapeDtypeStruct((NCHUNK * DSROWS, D), jnp.float32),
        ),
        compiler_params=_SC_PARAMS,
        mesh=plsc.VectorSubcoreMesh(core_axis_name="c", subcore_axis_name="s"),
        scratch_types=[
            pltpu.VMEM((32,), jnp.int32),
            pltpu.VMEM((B_PROC,), jnp.int32),
            pltpu.VMEM((B_PROC,), jnp.float32),
            pltpu.VMEM((B_PROC,), jnp.int32),
            pltpu.VMEM((B_PROC,), jnp.int32),
            pltpu.VMEM((B_PROC,), jnp.int32),
            pltpu.VMEM((H, B_PROC), jnp.float32),
            pltpu.VMEM((B_PROC, QW), jnp.float32),
            pltpu.VMEM((B_PROC, 2 * D), jnp.float32),
            pltpu.VMEM((B_PROC, D), jnp.float32),
            pltpu.VMEM((DSROWS, D), jnp.float32),
            pltpu.VMEM((DSROWS,), jnp.int32),
            pltpu.VMEM_SHARED((CHUNK, D), jnp.float32),
            pltpu.VMEM_SHARED((DSROWS, D), jnp.float32),
            pltpu.SemaphoreType.DMA,
            pltpu.SemaphoreType.DMA,
        ],
    )
    return fn(qaug, kv, bpk, bat, bcnt)


def _prep_layer(Wq, bq, Wk, bk, Wv, bv, We, Wskip, bskip, Wbeta):
    invc = 1.0 / jnp.sqrt(float(C))
    we_row = We[0]
    head_of = jnp.arange(D, dtype=jnp.int32) // C
    hsel = (head_of[:, None] == jnp.arange(H)[None, :]).astype(jnp.float32)
    wesel = hsel * we_row[:, None]

    Wqs = Wq * invc
    A = jnp.concatenate([Wqs, Wqs @ wesel, jnp.zeros((D, QW - D - H), jnp.float32)], axis=1)
    ba = jnp.concatenate([bq * invc, (bq * invc) @ wesel,
                          jnp.zeros((QW - D - H,), jnp.float32)])[None, :]
    B = jnp.concatenate([Wk, Wv], axis=1)
    bb = jnp.concatenate([bk, bv])[None, :]

    onehot_hd = (jnp.arange(H)[:, None] == head_of[None, :]).astype(jnp.float32)
    selden = jnp.zeros((8, D), jnp.float32).at[0:H, :].set(onehot_hd)
    sels = jnp.zeros((8, D), jnp.float32).at[H:2 * H, :].set(
        onehot_hd * we_row[None, :])

    wba = (Wbeta[0:D, 0] + Wbeta[2 * D:3 * D, 0])[None, :]
    wbb = (Wbeta[D:2 * D, 0] - Wbeta[2 * D:3 * D, 0])[None, :]
    return dict(A=A, ba=ba, B=B, bb=bb, Ws=Wskip, bs=bskip[None, :],
                selden=selden, sels=sels, wba=wba, wbb=wbb)


def _fold_encoder(p, enc_W, enc_b):
    q = dict(p)
    for mk, bk_ in (("A", "ba"), ("B", "bb"), ("Ws", "bs")):
        q[bk_] = (enc_b @ p[mk])[None, :] + p[bk_]
        q[mk] = enc_W @ p[mk]
    return q


def kernel(x, edge_index, edge_attr, batch, enc_W, enc_b,
           Wq0, bq0, Wk0, bk0, Wv0, bv0, We0, Wskip0, bskip0, Wbeta0, gamma0, beta0,
           Wq1, bq1, Wk1, bk1, Wv1, bv1, We1, Wskip1, bskip1, Wbeta1, gamma1, beta1,
           dec_W, dec_b, out_bias):
    src = jnp.pad(edge_index[0], (0, EPADV - E))
    dst = jnp.pad(edge_index[1], (0, EPADV - E))
    attr = jnp.pad(edge_attr[:, 0], (0, EPADV - E))
    batch3d = batch.reshape(NBLK, 1, BLK)

    p0 = _fold_encoder(_prep_layer(Wq0, bq0, Wk0, bk0, Wv0, bv0, We0,
                                   Wskip0, bskip0, Wbeta0), enc_W, enc_b)
    p1 = _prep_layer(Wq1, bq1, Wk1, bk1, Wv1, bv1, We1, Wskip1, bskip1, Wbeta1)

    bpk, bat, bcnt = _sc_bucket(src, dst, attr)

    qaug, kvp, xr = _tc_pre(x, p0["A"], p0["ba"], p0["B"], p0["bb"], p0["Ws"], p0["bs"])
    accn, accds = _sc_edge(qaug, kvp, bpk, bat, bcnt)
    ds8 = accds.reshape(NPAD, 8)
    h, stats = _tc_post1(accn, ds8, xr, p0["selden"], p0["sels"], p0["wba"], p0["wbb"])
    hn = _tc_post2_mid(h, stats, gamma0[None, :], beta0[None, :])

    qaug, kvp, xr = _tc_pre(hn, p1["A"], p1["ba"], p1["B"], p1["bb"], p1["Ws"], p1["bs"])
    accn, accds = _sc_edge(qaug, kvp, bpk, bat, bcnt)
    ds8 = accds.reshape(NPAD, 8)
    h, stats = _tc_post1(accn, ds8, xr, p1["selden"], p1["sels"], p1["wba"], p1["wbb"])
    pooled, cnts = _tc_post2_final(h, stats, gamma1[None, :], beta1[None, :], batch3d)

    wp = jnp.concatenate([dec_W, jnp.zeros((D, 7), jnp.float32)], axis=1)
    bp = jnp.broadcast_to((dec_b + out_bias)[:, None], (1, 8)).reshape(1, 8)
    y8 = _tc_dec(pooled, cnts, wp, bp)
    return y8[:, 0:1]
